# baseline (speedup 1.0000x reference)
import dataclasses
import functools

import jax
import jax.numpy as jnp
import numpy as np
from jax import lax
from jax._src import config as _jax_config
from jax.experimental import pallas as pl
from jax.experimental.pallas import tpu as pltpu
from jax.experimental.pallas import tpu_sc as plsc

N = 10000
E = 320000
D = 128
CH = 128
NC = 2
NS = 16
ROWS = 2560
EP = ROWS * CH
RPT = ROWS // (NC * NS)
PH = 2
RPP = RPT // PH
EPT = EP // (NC * NS)
NV = EPT // 16
H = 10240
HPT = H // NS
VL = 16


@functools.lru_cache(maxsize=None)
def _build_sc_kernels():
    mesh = plsc.VectorSubcoreMesh(core_axis_name="c", subcore_axis_name="s",
                                  num_cores=NC, num_subcores=NS)
    cp = pltpu.CompilerParams()
    if "needs_layout_passes" in pltpu.CompilerParams.__dataclass_fields__:
        cp = dataclasses.replace(cp, needs_layout_passes=False)


    @functools.partial(
        pl.kernel,
        out_type=jax.ShapeDtypeStruct((NC, H), jnp.float32),
        mesh=mesh,
        scratch_types=[
            pltpu.VMEM((RPT, 1, CH), jnp.int32),
            pltpu.VMEM((H,), jnp.float32),
            pltpu.VMEM((NS, HPT), jnp.float32),
            pltpu.VMEM((HPT,), jnp.float32),
            pltpu.VMEM_SHARED((NS, H), jnp.float32),
        ],
        compiler_params=cp,
    )
    def sc_deg(dst_hbm, out_hbm, dst_ts, hist_ts, red_ts, acc_ts, hist_sh):
        c = lax.axis_index("c")
        s = lax.axis_index("s")
        base = (c * jnp.int32(NS) + s) * jnp.int32(RPT)
        pltpu.sync_copy(dst_hbm.at[pl.ds(base, RPT)], dst_ts)
        zero16 = jnp.zeros((VL,), jnp.float32)
        one16 = jnp.ones((VL,), jnp.float32)

        @pl.loop(0, H // VL)
        def _(g):
            hist_ts[pl.ds(g * jnp.int32(VL), VL)] = zero16

        @pl.loop(0, RPT)
        def _(j):
            for k in range(CH // VL):
                plsc.addupdate_scatter(
                    hist_ts, [dst_ts[j, 0, pl.ds(k * VL, VL)]], one16)

        pltpu.sync_copy(hist_ts, hist_sh.at[s])
        plsc.subcore_barrier()
        pltpu.sync_copy(hist_sh.at[:, pl.ds(s * jnp.int32(HPT), HPT)], red_ts)

        @pl.loop(0, HPT // VL)
        def _(g):
            sl = pl.ds(g * jnp.int32(VL), VL)
            tot = red_ts[0, sl]
            for k in range(1, NS):
                tot = tot + red_ts[k, sl]
            acc_ts[sl] = tot

        pltpu.sync_copy(acc_ts, out_hbm.at[c, pl.ds(s * jnp.int32(HPT), HPT)])


    @functools.partial(
        pl.kernel,
        out_type=jax.ShapeDtypeStruct((NC, H, D), jnp.float32),
        mesh=mesh,
        scratch_types=[
            pltpu.VMEM((RPP, 1, CH), jnp.int32),
            pltpu.VMEM((RPP, 1, CH), jnp.int32),
            pltpu.VMEM((CH, D), jnp.float32),
            pltpu.VMEM((CH, D), jnp.float32),
            pltpu.VMEM_SHARED((H, D), jnp.float32),
            pltpu.SemaphoreType.DMA,
            pltpu.SemaphoreType.DMA,
            pltpu.SemaphoreType.DMA,
        ],
        compiler_params=cp,
    )
    def sc_edge(y_hbm, src_hbm, dst_hbm, zeros_hbm, out_hbm,
                src_ts, dst_ts, rows0, rows1, acc_sh, sem0, sem1, semz):
        c = lax.axis_index("c")
        s = lax.axis_index("s")
        base = (c * jnp.int32(NS) + s) * jnp.int32(RPT)
        hslc = pl.ds(s * jnp.int32(HPT), HPT)
        pltpu.async_copy(zeros_hbm.at[hslc], acc_sh.at[hslc], semz)

        for p in range(PH):
            pbase = base + jnp.int32(p * RPP)
            pltpu.sync_copy(src_hbm.at[pl.ds(pbase, RPP)], src_ts)
            pltpu.sync_copy(dst_hbm.at[pl.ds(pbase, RPP)], dst_ts)
            pltpu.async_copy(y_hbm.at[src_ts.at[0, 0]], rows0, sem0)
            if p == 0:
                pltpu.make_async_copy(zeros_hbm.at[hslc], acc_sh.at[hslc],
                                      semz).wait()
                plsc.subcore_barrier()

            @pl.loop(0, RPP // 2 - 1)
            def _(jj):
                j0 = jj * jnp.int32(2)
                j1 = j0 + jnp.int32(1)
                pltpu.async_copy(y_hbm.at[src_ts.at[j1, 0]], rows1, sem1)
                pltpu.make_async_copy(
                    y_hbm.at[src_ts.at[j0, 0]], rows0, sem0).wait()
                pltpu.sync_copy(rows0, acc_sh.at[dst_ts.at[j0, 0]], add=True)
                pltpu.async_copy(
                    y_hbm.at[src_ts.at[j0 + jnp.int32(2), 0]], rows0, sem0)
                pltpu.make_async_copy(
                    y_hbm.at[src_ts.at[j1, 0]], rows1, sem1).wait()
                pltpu.sync_copy(rows1, acc_sh.at[dst_ts.at[j1, 0]], add=True)

            ja = jnp.int32(RPP - 2)
            jb = jnp.int32(RPP - 1)
            pltpu.async_copy(y_hbm.at[src_ts.at[jb, 0]], rows1, sem1)
            pltpu.make_async_copy(y_hbm.at[src_ts.at[ja, 0]], rows0, sem0).wait()
            pltpu.sync_copy(rows0, acc_sh.at[dst_ts.at[ja, 0]], add=True)
            pltpu.make_async_copy(y_hbm.at[src_ts.at[jb, 0]], rows1, sem1).wait()
            pltpu.sync_copy(rows1, acc_sh.at[dst_ts.at[jb, 0]], add=True)

        plsc.subcore_barrier()
        pltpu.sync_copy(acc_sh.at[hslc], out_hbm.at[c, hslc])

    return sc_deg, sc_edge


BR = 2048
GRID = H // BR

def _dis_from(dp):
    deg = dp[0:1, :] + dp[1:2, :]
    dis = lax.rsqrt(deg + 1.0)
    return jnp.reshape(dis, (BR, 1))

_PREC = lax.Precision.DEFAULT

_row_spec = pl.BlockSpec((BR, D), lambda i: (i, 0))
_dp_spec = pl.BlockSpec((NC, BR), lambda i: (0, i))
_acc_spec = pl.BlockSpec((NC, BR, D), lambda i: (0, i, 0))
_w_spec = pl.BlockSpec((D, D), lambda i: (0, 0))
_b_spec = pl.BlockSpec((1, D), lambda i: (0, 0))


def _tc1a_body(x_ref, w_ref, o_ref):
    o_ref[...] = jnp.dot(x_ref[...], w_ref[...],
                         preferred_element_type=jnp.float32, precision=_PREC)


def _tc1b_body(xw_ref, dp_ref, y_ref):
    y_ref[...] = xw_ref[...] * _dis_from(dp_ref[...])


def _tc2_body(a_ref, y_ref, dp_ref, b_ref, w_ref, o_ref):
    dis = _dis_from(dp_ref[...])
    acc = a_ref[0] + a_ref[1]
    h = jnp.maximum(dis * (acc + y_ref[...]) + b_ref[...], 0.0)
    o_ref[...] = jnp.dot(h, w_ref[...], preferred_element_type=jnp.float32,
                         precision=_PREC) * dis


def _tc3_body(a_ref, y_ref, dp_ref, b_ref, x_ref, wh_ref, wx_ref, bl_ref,
              o_ref):
    dis = _dis_from(dp_ref[...])
    acc = a_ref[0] + a_ref[1]
    h2 = dis * (acc + y_ref[...]) + b_ref[...]
    z = (jnp.dot(h2, wh_ref[...], preferred_element_type=jnp.float32,
                 precision=_PREC)
         + jnp.dot(x_ref[...], wx_ref[...], preferred_element_type=jnp.float32,
                   precision=_PREC)
         + bl_ref[0, 0])
    o_ref[...] = jax.nn.sigmoid(z)


_tc1a = pl.pallas_call(
    _tc1a_body, out_shape=jax.ShapeDtypeStruct((N, D), jnp.float32),
    grid=(GRID,), in_specs=[_row_spec, _w_spec], out_specs=_row_spec)
_tc1b = pl.pallas_call(
    _tc1b_body, out_shape=jax.ShapeDtypeStruct((N, D), jnp.float32),
    grid=(GRID,), in_specs=[_row_spec, _dp_spec], out_specs=_row_spec)
_tc2 = pl.pallas_call(
    _tc2_body, out_shape=jax.ShapeDtypeStruct((N, D), jnp.float32),
    grid=(GRID,), in_specs=[_acc_spec, _row_spec, _dp_spec, _b_spec, _w_spec],
    out_specs=_row_spec)
_tc3 = pl.pallas_call(
    _tc3_body, out_shape=jax.ShapeDtypeStruct((N, 1), jnp.float32),
    grid=(GRID,),
    in_specs=[_acc_spec, _row_spec, _dp_spec, _b_spec, _row_spec,
              pl.BlockSpec((D, 1), lambda i: (0, 0)),
              pl.BlockSpec((D, 1), lambda i: (0, 0)),
              pl.BlockSpec((1, 1), lambda i: (0, 0))],
    out_specs=pl.BlockSpec((BR, 1), lambda i: (i, 0)))

_ZEROS_D = np.zeros((H, D), np.float32)
_NPAD = EP - E
_PAD_SRC = (np.arange(_NPAD, dtype=np.int32) % N)
_PAD_DST = (N + np.arange(_NPAD, dtype=np.int32) % (H - N)).astype(np.int32)


def kernel(feature, edge_index, W1, b1, W2, b2, Wlin, blin):
    with _jax_config.enable_x64(False):
        feature = feature.astype(jnp.float32)
        if edge_index.dtype == jnp.int64:
            ei32 = lax.bitcast_convert_type(edge_index, jnp.int32)
            src = ei32[0, :, 0]
            dst = ei32[1, :, 0]
        else:
            src = edge_index[0].astype(jnp.int32)
            dst = edge_index[1].astype(jnp.int32)
        src_p = jnp.concatenate(
            [src, jnp.asarray(_PAD_SRC)]).reshape(ROWS, 1, CH)
        dst_p = jnp.concatenate(
            [dst, jnp.asarray(_PAD_DST)]).reshape(ROWS, 1, CH)

        zeros_d = jnp.asarray(_ZEROS_D)

        sc_deg, sc_edge = _build_sc_kernels()
        degp = sc_deg(dst_p)

        xw1 = _tc1a(feature, W1.astype(jnp.float32))
        y1 = _tc1b(xw1, degp)
        a1 = sc_edge(y1, src_p, dst_p, zeros_d)
        y2 = _tc2(a1, y1, degp, b1.reshape(1, D).astype(jnp.float32),
                  W2.astype(jnp.float32))
        a2 = sc_edge(y2, src_p, dst_p, zeros_d)
        wl = Wlin.astype(jnp.float32)
        out = _tc3(a2, y2, degp, b2.reshape(1, D).astype(jnp.float32), feature,
                   wl[:D], wl[D:], blin.reshape(1, 1).astype(jnp.float32))
        res = out[:, 0]
    return res.astype(jnp.float64)


# speedup vs baseline: 1.0253x; 1.0253x over previous
import dataclasses
import functools

import jax
import jax.numpy as jnp
import numpy as np
from jax import lax
from jax._src import config as _jax_config
from jax.experimental import pallas as pl
from jax.experimental.pallas import tpu as pltpu
from jax.experimental.pallas import tpu_sc as plsc

N = 10000
E = 320000
D = 128
CH = 128
NC = 2
NS = 16
ROWS = 2560
EP = ROWS * CH
RPT = ROWS // (NC * NS)
PH = 2
RPP = RPT // PH
EPT = EP // (NC * NS)
NV = EPT // 16
H = 10240
HPT = H // NS
VL = 16


@functools.lru_cache(maxsize=None)
def _build_sc_kernels():
    mesh = plsc.VectorSubcoreMesh(core_axis_name="c", subcore_axis_name="s",
                                  num_cores=NC, num_subcores=NS)
    cp = pltpu.CompilerParams()
    if "needs_layout_passes" in pltpu.CompilerParams.__dataclass_fields__:
        cp = dataclasses.replace(cp, needs_layout_passes=False)


    @functools.partial(
        pl.kernel,
        out_type=jax.ShapeDtypeStruct((NC, H), jnp.float32),
        mesh=mesh,
        scratch_types=[
            pltpu.VMEM((RPT, 1, CH), jnp.int32),
            pltpu.VMEM((H,), jnp.float32),
            pltpu.VMEM((NS, HPT), jnp.float32),
            pltpu.VMEM((HPT,), jnp.float32),
            pltpu.VMEM_SHARED((NS, H), jnp.float32),
        ],
        compiler_params=cp,
    )
    def sc_deg(dst_hbm, out_hbm, dst_ts, hist_ts, red_ts, acc_ts, hist_sh):
        c = lax.axis_index("c")
        s = lax.axis_index("s")
        base = (c * jnp.int32(NS) + s) * jnp.int32(RPT)
        pltpu.sync_copy(dst_hbm.at[pl.ds(base, RPT)], dst_ts)
        zero16 = jnp.zeros((VL,), jnp.float32)
        one16 = jnp.ones((VL,), jnp.float32)

        @pl.loop(0, H // VL)
        def _(g):
            hist_ts[pl.ds(g * jnp.int32(VL), VL)] = zero16

        @pl.loop(0, RPT)
        def _(j):
            for k in range(CH // VL):
                plsc.addupdate_scatter(
                    hist_ts, [dst_ts[j, 0, pl.ds(k * VL, VL)]], one16)

        pltpu.sync_copy(hist_ts, hist_sh.at[s])
        plsc.subcore_barrier()
        pltpu.sync_copy(hist_sh.at[:, pl.ds(s * jnp.int32(HPT), HPT)], red_ts)

        @pl.loop(0, HPT // VL)
        def _(g):
            sl = pl.ds(g * jnp.int32(VL), VL)
            tot = red_ts[0, sl]
            for k in range(1, NS):
                tot = tot + red_ts[k, sl]
            acc_ts[sl] = tot

        pltpu.sync_copy(acc_ts, out_hbm.at[c, pl.ds(s * jnp.int32(HPT), HPT)])


    @functools.partial(
        pl.kernel,
        out_type=jax.ShapeDtypeStruct((NC, H, D), jnp.float32),
        mesh=mesh,
        scratch_types=[
            pltpu.VMEM((RPP, 1, CH), jnp.int32),
            pltpu.VMEM((RPP, 1, CH), jnp.int32),
            pltpu.VMEM((CH, D), jnp.float32),
            pltpu.VMEM((CH, D), jnp.float32),
            pltpu.VMEM_SHARED((H, D), jnp.float32),
            pltpu.SemaphoreType.DMA,
            pltpu.SemaphoreType.DMA,
            pltpu.SemaphoreType.DMA,
        ],
        compiler_params=cp,
    )
    def sc_edge(y_hbm, src_hbm, dst_hbm, zeros_hbm, out_hbm,
                src_ts, dst_ts, rows0, rows1, acc_sh, sem0, sem1, semz):
        c = lax.axis_index("c")
        s = lax.axis_index("s")
        base = (c * jnp.int32(NS) + s) * jnp.int32(RPT)
        hslc = pl.ds(s * jnp.int32(HPT), HPT)
        pltpu.async_copy(zeros_hbm.at[hslc], acc_sh.at[hslc], semz)

        for p in range(PH):
            pbase = base + jnp.int32(p * RPP)
            pltpu.sync_copy(src_hbm.at[pl.ds(pbase, RPP)], src_ts)
            pltpu.sync_copy(dst_hbm.at[pl.ds(pbase, RPP)], dst_ts)
            pltpu.async_copy(y_hbm.at[src_ts.at[0, 0]], rows0, sem0)
            if p == 0:
                pltpu.make_async_copy(zeros_hbm.at[hslc], acc_sh.at[hslc],
                                      semz).wait()
                plsc.subcore_barrier()

            @pl.loop(0, RPP // 2 - 1)
            def _(jj):
                j0 = jj * jnp.int32(2)
                j1 = j0 + jnp.int32(1)
                pltpu.async_copy(y_hbm.at[src_ts.at[j1, 0]], rows1, sem1)
                pltpu.make_async_copy(
                    y_hbm.at[src_ts.at[j0, 0]], rows0, sem0).wait()
                pltpu.sync_copy(rows0, acc_sh.at[dst_ts.at[j0, 0]], add=True)
                pltpu.async_copy(
                    y_hbm.at[src_ts.at[j0 + jnp.int32(2), 0]], rows0, sem0)
                pltpu.make_async_copy(
                    y_hbm.at[src_ts.at[j1, 0]], rows1, sem1).wait()
                pltpu.sync_copy(rows1, acc_sh.at[dst_ts.at[j1, 0]], add=True)

            ja = jnp.int32(RPP - 2)
            jb = jnp.int32(RPP - 1)
            pltpu.async_copy(y_hbm.at[src_ts.at[jb, 0]], rows1, sem1)
            pltpu.make_async_copy(y_hbm.at[src_ts.at[ja, 0]], rows0, sem0).wait()
            pltpu.sync_copy(rows0, acc_sh.at[dst_ts.at[ja, 0]], add=True)
            pltpu.make_async_copy(y_hbm.at[src_ts.at[jb, 0]], rows1, sem1).wait()
            pltpu.sync_copy(rows1, acc_sh.at[dst_ts.at[jb, 0]], add=True)

        plsc.subcore_barrier()
        pltpu.sync_copy(acc_sh.at[hslc], out_hbm.at[c, hslc])

    return sc_deg, sc_edge


BR = 2048
GRID = H // BR

def _dis_from(dp):
    deg = dp[0:1, :] + dp[1:2, :]
    dis = lax.rsqrt(deg + 1.0)
    return jnp.reshape(dis, (BR, 1))

_PREC = lax.Precision.DEFAULT

_row_spec = pl.BlockSpec((BR, D), lambda i: (i, 0))
_dp_spec = pl.BlockSpec((NC, BR), lambda i: (0, i))
_acc_spec = pl.BlockSpec((NC, BR, D), lambda i: (0, i, 0))
_w_spec = pl.BlockSpec((D, D), lambda i: (0, 0))
_b_spec = pl.BlockSpec((1, D), lambda i: (0, 0))


def _tc1a_body(x_ref, w_ref, o_ref):
    o_ref[...] = jnp.dot(x_ref[...], w_ref[...],
                         preferred_element_type=jnp.float32, precision=_PREC)


def _tc1b_body(xw_ref, dp_ref, y_ref):
    y_ref[...] = xw_ref[...] * _dis_from(dp_ref[...])


def _tc2_body(a_ref, y_ref, dp_ref, b_ref, w_ref, o_ref):
    dis = _dis_from(dp_ref[...])
    acc = a_ref[0] + a_ref[1]
    h = jnp.maximum(dis * (acc + y_ref[...]) + b_ref[...], 0.0)
    o_ref[...] = jnp.dot(h, w_ref[...], preferred_element_type=jnp.float32,
                         precision=_PREC) * dis


def _tc3_body(a_ref, y_ref, dp_ref, b_ref, x_ref, wh_ref, wx_ref, bl_ref,
              o_ref):
    dis = _dis_from(dp_ref[...])
    acc = a_ref[0] + a_ref[1]
    h2 = dis * (acc + y_ref[...]) + b_ref[...]
    z = (jnp.dot(h2, wh_ref[...], preferred_element_type=jnp.float32,
                 precision=_PREC)
         + jnp.dot(x_ref[...], wx_ref[...], preferred_element_type=jnp.float32,
                   precision=_PREC)
         + bl_ref[0, 0])
    o_ref[...] = jax.nn.sigmoid(z)


_tc1a = pl.pallas_call(
    _tc1a_body, out_shape=jax.ShapeDtypeStruct((N, D), jnp.float32),
    grid=(GRID,), in_specs=[_row_spec, _w_spec], out_specs=_row_spec)
_tc1b = pl.pallas_call(
    _tc1b_body, out_shape=jax.ShapeDtypeStruct((N, D), jnp.float32),
    grid=(GRID,), in_specs=[_row_spec, _dp_spec], out_specs=_row_spec)
_tc2 = pl.pallas_call(
    _tc2_body, out_shape=jax.ShapeDtypeStruct((N, D), jnp.float32),
    grid=(GRID,), in_specs=[_acc_spec, _row_spec, _dp_spec, _b_spec, _w_spec],
    out_specs=_row_spec)
_tc3 = pl.pallas_call(
    _tc3_body, out_shape=jax.ShapeDtypeStruct((N, 1), jnp.float32),
    grid=(GRID,),
    in_specs=[_acc_spec, _row_spec, _dp_spec, _b_spec, _row_spec,
              pl.BlockSpec((D, 1), lambda i: (0, 0)),
              pl.BlockSpec((D, 1), lambda i: (0, 0)),
              pl.BlockSpec((1, 1), lambda i: (0, 0))],
    out_specs=pl.BlockSpec((BR, 1), lambda i: (i, 0)))

_ZEROS_D = np.zeros((H, D), np.float32)
_NPAD = EP - E
_PAD_SRC = (np.arange(_NPAD, dtype=np.int32) % N)
_PAD_DST = (N + np.arange(_NPAD, dtype=np.int32) % (H - N)).astype(np.int32)


def kernel(feature, edge_index, W1, b1, W2, b2, Wlin, blin):
    with _jax_config.enable_x64(False):
        feature = feature.astype(jnp.float32)
        src = edge_index[0].astype(jnp.int32)
        dst = edge_index[1].astype(jnp.int32)
        src_p = jnp.concatenate(
            [src, jnp.asarray(_PAD_SRC)]).reshape(ROWS, 1, CH)
        dst_p = jnp.concatenate(
            [dst, jnp.asarray(_PAD_DST)]).reshape(ROWS, 1, CH)

        zeros_d = jnp.asarray(_ZEROS_D)

        sc_deg, sc_edge = _build_sc_kernels()
        degp = sc_deg(dst_p)

        xw1 = _tc1a(feature, W1.astype(jnp.float32))
        y1 = _tc1b(xw1, degp)
        a1 = sc_edge(y1, src_p, dst_p, zeros_d)
        y2 = _tc2(a1, y1, degp, b1.reshape(1, D).astype(jnp.float32),
                  W2.astype(jnp.float32))
        a2 = sc_edge(y2, src_p, dst_p, zeros_d)
        wl = Wlin.astype(jnp.float32)
        out = _tc3(a2, y2, degp, b2.reshape(1, D).astype(jnp.float32), feature,
                   wl[:D], wl[D:], blin.reshape(1, 1).astype(jnp.float32))
        res = out[:, 0]
    return res.astype(jnp.float64)
